# baseline (speedup 1.0000x reference)
import functools

import jax
import jax.numpy as jnp
from jax import lax
from jax.experimental import pallas as pl
from jax.experimental.pallas import tpu as pltpu
from jax.experimental.pallas import tpu_sc as plsc

B, L, V, D, H, C = 1024, 200, 50003, 300, 256, 3
NC, NS = 2, 16
NW = NC * NS
BPW = B // NW
CH = 40
CPE = L // CH
NCH = BPW * CPE
NBUF = 5
GRP = CPE
EPG = 1
TPAD = 384
DP = 304
NV = DP // 16
CP = 128


def _sc_gather_sum(table, idx3):
    mesh = plsc.VectorSubcoreMesh(
        core_axis_name="c", subcore_axis_name="s",
        num_cores=NC, num_subcores=NS)

    @functools.partial(
        pl.kernel,
        out_type=jax.ShapeDtypeStruct((B, DP), jnp.float32),
        mesh=mesh,
        scratch_types=[
            pltpu.VMEM((NCH, CH), jnp.int32),
            pltpu.VMEM((NBUF, CH, TPAD), jnp.float32),
            pltpu.VMEM((BPW, DP), jnp.float32),
            [pltpu.SemaphoreType.DMA] * NBUF,
        ],
        compiler_params=pltpu.CompilerParams(use_tc_tiling_on_sc=True),
    )
    def k(table_hbm, idx_hbm, out_hbm, idx_v, bufs, out_v, sems):
        w = lax.axis_index("s") * NC + lax.axis_index("c")
        pltpu.sync_copy(idx_hbm.at[w], idx_v)

        def fire(g, slot):
            pltpu.make_async_copy(
                table_hbm.at[idx_v.at[g]], bufs.at[slot], sems[slot]).start()

        def drain(g, slot):
            pltpu.make_async_copy(
                table_hbm.at[idx_v.at[g]], bufs.at[slot], sems[slot]).wait()

        for s in range(NBUF):
            fire(s, s)

        def body(e4, carry):
            g0 = e4 * GRP
            accs = None
            for k in range(GRP):
                slot = k % NBUF
                g = g0 + k
                drain(g, slot)
                if k % CPE == 0:
                    accs = tuple(jnp.zeros((16,), jnp.float32)
                                 for _ in range(NV))

                def row_red(r, accs, slot=slot):
                    accs = list(accs)
                    for j in range(NV):
                        accs[j] = accs[j] + bufs[slot, r, pl.ds(j * 16, 16)]
                    return tuple(accs)

                accs = lax.fori_loop(0, CH, row_red, accs, unroll=2)

                @pl.when(g + NBUF < NCH)
                def _(g=g, slot=slot):
                    fire(g + NBUF, slot)

                if k % CPE == CPE - 1:
                    e_loc = e4 * EPG + k // CPE
                    for j in range(NV):
                        out_v[e_loc, pl.ds(j * 16, 16)] = accs[j]
            return carry

        lax.fori_loop(0, BPW // EPG, body, 0)
        pltpu.sync_copy(out_v, out_hbm.at[pl.ds(w * BPW, BPW)])

    return k(table, idx3)


def _tc_pad_table(table_t):
    vb = 2048
    grid = (pl.cdiv(V, vb),)

    def body(x_ref, o_ref):
        o_ref[...] = jnp.pad(x_ref[...].T, ((0, 0), (0, TPAD - D)))

    return pl.pallas_call(
        body,
        grid=grid,
        in_specs=[pl.BlockSpec((D, vb), lambda i: (0, i))],
        out_specs=pl.BlockSpec((vb, TPAD), lambda i: (i, 0)),
        out_shape=jax.ShapeDtypeStruct((V, TPAD), jnp.float32),
    )(table_t)


def _tc_head(sums, targets2d, w_h, b_h2, w_o, b_o2):
    def body(x_ref, t_ref, wh_ref, bh_ref, wo_ref, bo_ref, out_ref):
        x = x_ref[...] * jnp.float32(1.0 / L)
        conv = jnp.tanh(
            jnp.dot(x, wh_ref[...], preferred_element_type=jnp.float32,
                    precision=lax.Precision.HIGHEST)
            + bh_ref[...])
        logits = (jnp.dot(conv, wo_ref[...], preferred_element_type=jnp.float32,
                          precision=lax.Precision.HIGHEST)
                  + bo_ref[...])
        col = lax.broadcasted_iota(jnp.int32, logits.shape, 1)
        ml = jnp.where(col < C, logits, jnp.float32(-1e30))
        m = jnp.max(ml, axis=1, keepdims=True)
        s = jnp.sum(jnp.exp(ml - m), axis=1, keepdims=True)
        lse = m + jnp.log(s)
        tsel = jnp.sum(jnp.where(col == t_ref[...], ml, 0.0), axis=1,
                       keepdims=True)
        out_ref[0, 0] = jnp.sum(lse - tsel)

    return pl.pallas_call(
        body,
        out_shape=jax.ShapeDtypeStruct((1, 1), jnp.float32),
        out_specs=pl.BlockSpec(memory_space=pltpu.SMEM),
    )(sums, targets2d, w_h, b_h2, w_o, b_o2)


def kernel(input_words, targets, emb_table, W_h, b_h, W_o, b_o):
    idx3 = input_words.reshape(NW, NCH, CH)
    table_p = _tc_pad_table(emb_table.T)
    sums = _sc_gather_sum(table_p, idx3)
    w_h_p = jnp.zeros((DP, H), W_h.dtype).at[:D].set(W_h)
    w_o_p = jnp.zeros((H, CP), W_o.dtype).at[:, :C].set(W_o)
    b_o_p = jnp.zeros((1, CP), b_o.dtype).at[0, :C].set(b_o)
    loss = _tc_head(sums, targets.reshape(B, 1).astype(jnp.int32),
                    w_h_p, b_h.reshape(1, H), w_o_p, b_o_p)
    return loss[0, 0]


# speedup vs baseline: 1.0220x; 1.0220x over previous
import functools

import jax
import jax.numpy as jnp
from jax import lax
from jax.experimental import pallas as pl
from jax.experimental.pallas import tpu as pltpu
from jax.experimental.pallas import tpu_sc as plsc

B, L, V, D, H, C = 1024, 200, 50003, 300, 256, 3
NC, NS = 2, 16
NW = NC * NS
BPW = B // NW
CH = 40
CPE = L // CH
NCH = BPW * CPE
NBUF = 5
GRP = CPE
EPG = 1
TPAD = 384
DP = 304
NV = DP // 16
CP = 128


def _sc_gather_sum(table, idx3):
    mesh = plsc.VectorSubcoreMesh(
        core_axis_name="c", subcore_axis_name="s",
        num_cores=NC, num_subcores=NS)

    @functools.partial(
        pl.kernel,
        out_type=jax.ShapeDtypeStruct((B, DP), jnp.float32),
        mesh=mesh,
        scratch_types=[
            pltpu.VMEM((NCH, CH), jnp.int32),
            pltpu.VMEM((NBUF, CH, TPAD), jnp.float32),
            pltpu.VMEM((BPW, DP), jnp.float32),
            [pltpu.SemaphoreType.DMA] * NBUF,
        ],
        compiler_params=pltpu.CompilerParams(use_tc_tiling_on_sc=True),
    )
    def k(table_hbm, idx_hbm, out_hbm, idx_v, bufs, out_v, sems):
        w = lax.axis_index("s") * NC + lax.axis_index("c")
        pltpu.sync_copy(idx_hbm.at[w], idx_v)

        def fire(g, slot):
            pltpu.make_async_copy(
                table_hbm.at[idx_v.at[g]], bufs.at[slot], sems[slot]).start()

        def drain(g, slot):
            pltpu.make_async_copy(
                table_hbm.at[idx_v.at[g]], bufs.at[slot], sems[slot]).wait()

        for s in range(NBUF):
            fire(s, s)

        def body(e4, carry):
            g0 = e4 * GRP
            accs = None
            for k in range(GRP):
                slot = k % NBUF
                g = g0 + k
                drain(g, slot)
                if k % CPE == 0:
                    accs = tuple(jnp.zeros((16,), jnp.float32)
                                 for _ in range(NV))

                def row_red(r, accs, slot=slot):
                    accs = list(accs)
                    for j in range(NV):
                        accs[j] = accs[j] + bufs[slot, r, pl.ds(j * 16, 16)]
                    return tuple(accs)

                accs = lax.fori_loop(0, CH, row_red, accs, unroll=2)

                @pl.when(g + NBUF < NCH)
                def _(g=g, slot=slot):
                    fire(g + NBUF, slot)

                if k % CPE == CPE - 1:
                    e_loc = e4 * EPG + k // CPE
                    for j in range(NV):
                        out_v[e_loc, pl.ds(j * 16, 16)] = accs[j]
            return carry

        lax.fori_loop(0, BPW // EPG, body, 0)
        pltpu.sync_copy(out_v, out_hbm.at[pl.ds(w * BPW, BPW)])

    return k(table, idx3)


def _tc_pad_table(table_t):
    vb = 4096
    grid = (pl.cdiv(V, vb),)

    def body(x_ref, o_ref):
        o_ref[...] = jnp.pad(x_ref[...].T, ((0, 0), (0, TPAD - D)))

    return pl.pallas_call(
        body,
        grid=grid,
        in_specs=[pl.BlockSpec((D, vb), lambda i: (0, i))],
        out_specs=pl.BlockSpec((vb, TPAD), lambda i: (i, 0)),
        out_shape=jax.ShapeDtypeStruct((V, TPAD), jnp.float32),
    )(table_t)


def _tc_head(sums, targets2d, w_h, b_h2, w_o, b_o2):
    def body(x_ref, t_ref, wh_ref, bh_ref, wo_ref, bo_ref, out_ref):
        x = x_ref[...] * jnp.float32(1.0 / L)
        conv = jnp.tanh(
            jnp.dot(x, wh_ref[...], preferred_element_type=jnp.float32,
                    precision=lax.Precision.HIGHEST)
            + bh_ref[...])
        logits = (jnp.dot(conv, wo_ref[...], preferred_element_type=jnp.float32,
                          precision=lax.Precision.HIGHEST)
                  + bo_ref[...])
        col = lax.broadcasted_iota(jnp.int32, logits.shape, 1)
        ml = jnp.where(col < C, logits, jnp.float32(-1e30))
        m = jnp.max(ml, axis=1, keepdims=True)
        s = jnp.sum(jnp.exp(ml - m), axis=1, keepdims=True)
        lse = m + jnp.log(s)
        tsel = jnp.sum(jnp.where(col == t_ref[...], ml, 0.0), axis=1,
                       keepdims=True)
        out_ref[0, 0] = jnp.sum(lse - tsel)

    return pl.pallas_call(
        body,
        out_shape=jax.ShapeDtypeStruct((1, 1), jnp.float32),
        out_specs=pl.BlockSpec(memory_space=pltpu.SMEM),
    )(sums, targets2d, w_h, b_h2, w_o, b_o2)


def kernel(input_words, targets, emb_table, W_h, b_h, W_o, b_o):
    idx3 = input_words.reshape(NW, NCH, CH)
    table_p = _tc_pad_table(emb_table.T)
    sums = _sc_gather_sum(table_p, idx3)
    w_h_p = jnp.zeros((DP, H), W_h.dtype).at[:D].set(W_h)
    w_o_p = jnp.zeros((H, CP), W_o.dtype).at[:, :C].set(W_o)
    b_o_p = jnp.zeros((1, CP), b_o.dtype).at[0, :C].set(b_o)
    loss = _tc_head(sums, targets.reshape(B, 1).astype(jnp.int32),
                    w_h_p, b_h.reshape(1, H), w_o_p, b_o_p)
    return loss[0, 0]
